# baseline (speedup 1.0000x reference)
import functools

import jax
import jax.numpy as jnp
from jax import lax
from jax.experimental import pallas as pl
from jax.experimental.pallas import tpu as pltpu

B, C, H, W = 8, 384, 224, 224
HID, E, K = 96, 64, 2
S = H * W
HB = 16
NH = H // HB


def _body(x_ref, w1_ref, b1_ref, w2_ref, b2_ref,
          idx_ref, val_ref, probs_ref, part_ref):
    b = pl.program_id(0)
    hb = pl.program_id(1)
    psum = jnp.sum(x_ref[...], axis=(1, 2))

    @pl.when(hb == 0)
    def _init():
        part_ref[pl.ds(b, 1), :] = psum

    @pl.when(hb != 0)
    def _acc():
        part_ref[pl.ds(b, 1), :] += psum

    @pl.when((b == B - 1) & (hb == NH - 1))
    def _router():
        h = part_ref[...] * (1.0 / S)
        hid = lax.dot_general(h, w1_ref[...], (((1,), (1,)), ((), ())),
                              preferred_element_type=jnp.float32)
        hid = jnp.maximum(hid + b1_ref[...], 0.0)
        logits = lax.dot_general(hid, w2_ref[...], (((1,), (1,)), ((), ())),
                                 preferred_element_type=jnp.float32)
        logits = logits + b2_ref[...]
        m = jnp.max(logits, axis=1, keepdims=True)
        e = jnp.exp(logits - m)
        p = e / jnp.sum(e, axis=1, keepdims=True)
        probs_ref[...] = p
        iota = lax.broadcasted_iota(jnp.int32, p.shape, 1)
        m1 = jnp.max(p, axis=1, keepdims=True)
        i1 = jnp.min(jnp.where(p == m1, iota, E), axis=1, keepdims=True)
        p2 = jnp.where(iota == i1, -jnp.inf, p)
        m2 = jnp.max(p2, axis=1, keepdims=True)
        i2 = jnp.min(jnp.where(p2 == m2, iota, E), axis=1, keepdims=True)
        val_ref[...] = jnp.concatenate([m1, m2], axis=1)
        idx_ref[...] = jnp.concatenate([i1, i2], axis=1)


@jax.jit
def kernel(x, W1, b1, W2, b2):
    xt = jnp.transpose(x, (0, 2, 3, 1))
    b1r = b1.reshape(1, HID)
    b2r = b2.reshape(1, E)

    out = pl.pallas_call(
        _body,
        grid=(B, NH),
        in_specs=[pl.BlockSpec((1, HB, W, C), lambda b, hb: (b, hb, 0, 0)),
                  pl.BlockSpec((HID, C), lambda b, hb: (0, 0)),
                  pl.BlockSpec((1, HID), lambda b, hb: (0, 0)),
                  pl.BlockSpec((E, HID), lambda b, hb: (0, 0)),
                  pl.BlockSpec((1, E), lambda b, hb: (0, 0))],
        out_specs=[
            pl.BlockSpec((B, K), lambda b, hb: (0, 0)),
            pl.BlockSpec((B, K), lambda b, hb: (0, 0)),
            pl.BlockSpec((B, E), lambda b, hb: (0, 0)),
        ],
        out_shape=[
            jax.ShapeDtypeStruct((B, K), jnp.int32),
            jax.ShapeDtypeStruct((B, K), jnp.float32),
            jax.ShapeDtypeStruct((B, E), jnp.float32),
        ],
        scratch_shapes=[pltpu.VMEM((B, C), jnp.float32)],
        compiler_params=pltpu.CompilerParams(
            dimension_semantics=("arbitrary", "arbitrary"),
        ),
    )(xt, W1, b1r, W2, b2r)
    topk_idx, topk_val, probs = out
    return (topk_idx, topk_val, probs)


# speedup vs baseline: 1.0208x; 1.0208x over previous
import functools

import jax
import jax.numpy as jnp
from jax import lax
from jax.experimental import pallas as pl
from jax.experimental.pallas import tpu as pltpu

B, C, H, W = 8, 384, 224, 224
HID, E, K = 96, 64, 2
S = H * W
RAMP = (4, 8, 16)
R0 = sum(RAMP)
HB = 28
NST = (B * H - R0) // HB
NBUF = 4
LA = 3


def _body(x_ref, w1_ref, b1_ref, w2_ref, b2_ref,
          idx_ref, val_ref, probs_ref,
          rb0, rb1, rb2, bufs, part_ref, rsems, sems):
    rbufs = (rb0, rb1, rb2)

    def src(r0, n):
        return x_ref.at[pl.ds(r0, n)]

    r = 0
    for i, n in enumerate(RAMP):
        pltpu.make_async_copy(src(r, n), rbufs[i], rsems.at[i]).start()
        r += n
    for t in range(LA):
        pltpu.make_async_copy(src(R0 + t * HB, HB), bufs.at[t], sems.at[t]).start()

    part_ref[...] = jnp.zeros((B, C), jnp.float32)

    r = 0
    for i, n in enumerate(RAMP):
        pltpu.make_async_copy(src(r, n), rbufs[i], rsems.at[i]).wait()
        part_ref[pl.ds(0, 1), :] += jnp.sum(rbufs[i][...], axis=(0, 1))[None, :]
        r += n

    def step(t, _):
        k = lax.rem(t, NBUF)
        r0 = R0 + t * HB
        pltpu.make_async_copy(src(r0, HB), bufs.at[k], sems.at[k]).wait()

        @pl.when(t + LA < NST)
        def _issue():
            kn = lax.rem(t + LA, NBUF)
            pltpu.make_async_copy(
                src(R0 + (t + LA) * HB, HB), bufs.at[kn], sems.at[kn]).start()

        psum = jnp.sum(bufs[k], axis=(0, 1))[None, :]
        b = r0 // H
        part_ref[pl.ds(b, 1), :] += psum
        return 0

    lax.fori_loop(0, NST, step, 0)

    h = part_ref[...] * (1.0 / S)
    hid = lax.dot_general(h, w1_ref[...], (((1,), (1,)), ((), ())),
                          preferred_element_type=jnp.float32)
    hid = jnp.maximum(hid + b1_ref[...], 0.0)
    logits = lax.dot_general(hid, w2_ref[...], (((1,), (1,)), ((), ())),
                             preferred_element_type=jnp.float32)
    logits = logits + b2_ref[...]
    m = jnp.max(logits, axis=1, keepdims=True)
    e = jnp.exp(logits - m)
    p = e / jnp.sum(e, axis=1, keepdims=True)
    probs_ref[...] = p
    iota = lax.broadcasted_iota(jnp.int32, p.shape, 1)
    m1 = jnp.max(p, axis=1, keepdims=True)
    i1 = jnp.min(jnp.where(p == m1, iota, E), axis=1, keepdims=True)
    p2 = jnp.where(iota == i1, -jnp.inf, p)
    m2 = jnp.max(p2, axis=1, keepdims=True)
    i2 = jnp.min(jnp.where(p2 == m2, iota, E), axis=1, keepdims=True)
    val_ref[...] = jnp.concatenate([m1, m2], axis=1)
    idx_ref[...] = jnp.concatenate([i1, i2], axis=1)


@jax.jit
def kernel(x, W1, b1, W2, b2):
    xt = jnp.transpose(x, (0, 2, 3, 1))
    xf = xt.reshape(B * H, W, C)
    b1r = b1.reshape(1, HID)
    b2r = b2.reshape(1, E)

    out = pl.pallas_call(
        _body,
        in_specs=[pl.BlockSpec(memory_space=pl.ANY),
                  pl.BlockSpec((HID, C), lambda: (0, 0)),
                  pl.BlockSpec((1, HID), lambda: (0, 0)),
                  pl.BlockSpec((E, HID), lambda: (0, 0)),
                  pl.BlockSpec((1, E), lambda: (0, 0))],
        out_specs=[
            pl.BlockSpec((B, K), lambda: (0, 0)),
            pl.BlockSpec((B, K), lambda: (0, 0)),
            pl.BlockSpec((B, E), lambda: (0, 0)),
        ],
        out_shape=[
            jax.ShapeDtypeStruct((B, K), jnp.int32),
            jax.ShapeDtypeStruct((B, K), jnp.float32),
            jax.ShapeDtypeStruct((B, E), jnp.float32),
        ],
        scratch_shapes=[
            pltpu.VMEM((RAMP[0], W, C), jnp.float32),
            pltpu.VMEM((RAMP[1], W, C), jnp.float32),
            pltpu.VMEM((RAMP[2], W, C), jnp.float32),
            pltpu.VMEM((NBUF, HB, W, C), jnp.float32),
            pltpu.VMEM((B, C), jnp.float32),
            pltpu.SemaphoreType.DMA((len(RAMP),)),
            pltpu.SemaphoreType.DMA((NBUF,)),
        ],
    )(xf, W1, b1r, W2, b2r)
    topk_idx, topk_val, probs = out
    return (topk_idx, topk_val, probs)


# speedup vs baseline: 1.0227x; 1.0019x over previous
import functools

import jax
import jax.numpy as jnp
from jax import lax
from jax.experimental import pallas as pl
from jax.experimental.pallas import tpu as pltpu

B, C, H, W = 8, 384, 224, 224
HID, E, K = 96, 64, 2
S = H * W
HB = 28
NH = H // HB


def _body(x_ref, w1_ref, b1_ref, w2_ref, b2_ref,
          idx_ref, val_ref, probs_ref, part_ref):
    b = pl.program_id(0)
    hb = pl.program_id(1)
    psum = jnp.sum(x_ref[...], axis=(1, 2))

    @pl.when(hb == 0)
    def _init():
        part_ref[pl.ds(b, 1), :] = psum

    @pl.when(hb != 0)
    def _acc():
        part_ref[pl.ds(b, 1), :] += psum

    @pl.when((b == B - 1) & (hb == NH - 1))
    def _router():
        h = part_ref[...] * (1.0 / S)
        hid = lax.dot_general(h, w1_ref[...], (((1,), (1,)), ((), ())),
                              preferred_element_type=jnp.float32)
        hid = jnp.maximum(hid + b1_ref[...], 0.0)
        logits = lax.dot_general(hid, w2_ref[...], (((1,), (1,)), ((), ())),
                                 preferred_element_type=jnp.float32)
        logits = logits + b2_ref[...]
        m = jnp.max(logits, axis=1, keepdims=True)
        e = jnp.exp(logits - m)
        p = e / jnp.sum(e, axis=1, keepdims=True)
        probs_ref[...] = p
        iota = lax.broadcasted_iota(jnp.int32, p.shape, 1)
        m1 = jnp.max(p, axis=1, keepdims=True)
        i1 = jnp.min(jnp.where(p == m1, iota, E), axis=1, keepdims=True)
        p2 = jnp.where(iota == i1, -jnp.inf, p)
        m2 = jnp.max(p2, axis=1, keepdims=True)
        i2 = jnp.min(jnp.where(p2 == m2, iota, E), axis=1, keepdims=True)
        val_ref[...] = jnp.concatenate([m1, m2], axis=1)
        idx_ref[...] = jnp.concatenate([i1, i2], axis=1)


@jax.jit
def kernel(x, W1, b1, W2, b2):
    xt = jnp.transpose(x, (0, 2, 3, 1))
    b1r = b1.reshape(1, HID)
    b2r = b2.reshape(1, E)

    out = pl.pallas_call(
        _body,
        grid=(B, NH),
        in_specs=[pl.BlockSpec((1, HB, W, C), lambda b, hb: (b, hb, 0, 0)),
                  pl.BlockSpec((HID, C), lambda b, hb: (0, 0)),
                  pl.BlockSpec((1, HID), lambda b, hb: (0, 0)),
                  pl.BlockSpec((E, HID), lambda b, hb: (0, 0)),
                  pl.BlockSpec((1, E), lambda b, hb: (0, 0))],
        out_specs=[
            pl.BlockSpec((B, K), lambda b, hb: (0, 0)),
            pl.BlockSpec((B, K), lambda b, hb: (0, 0)),
            pl.BlockSpec((B, E), lambda b, hb: (0, 0)),
        ],
        out_shape=[
            jax.ShapeDtypeStruct((B, K), jnp.int32),
            jax.ShapeDtypeStruct((B, K), jnp.float32),
            jax.ShapeDtypeStruct((B, E), jnp.float32),
        ],
        scratch_shapes=[pltpu.VMEM((B, C), jnp.float32)],
        compiler_params=pltpu.CompilerParams(
            dimension_semantics=("arbitrary", "arbitrary"),
        ),
    )(xt, W1, b1r, W2, b2r)
    topk_idx, topk_val, probs = out
    return (topk_idx, topk_val, probs)
